# baseline (speedup 1.0000x reference)
import jax
import jax.numpy as jnp
from jax.experimental import pallas as pl
from jax.experimental.pallas import tpu as pltpu

N = 4096
IN_F = 64
OUT_F = 64
BLOCK_ROWS = 512
HALF = N // 2


def _gcn_kernel(inp0_ref, inp1_ref, adj0_ref, adj1_ref, w_ref, b_ref, out_ref):
    t = jnp.dot(adj0_ref[...], inp0_ref[...], preferred_element_type=jnp.float32)
    t = t + jnp.dot(adj1_ref[...], inp1_ref[...], preferred_element_type=jnp.float32)
    out_ref[...] = (
        jnp.dot(t, w_ref[...], preferred_element_type=jnp.float32) + b_ref[...]
    )


def kernel(input, adj, W, b):
    b2 = b.reshape(1, OUT_F)
    inp0 = input[:HALF]
    inp1 = input[HALF:]
    adj0 = adj[:, :HALF]
    adj1 = adj[:, HALF:]
    grid = (N // BLOCK_ROWS,)
    return pl.pallas_call(
        _gcn_kernel,
        grid=grid,
        in_specs=[
            pl.BlockSpec((HALF, IN_F), lambda i: (0, 0)),
            pl.BlockSpec((HALF, IN_F), lambda i: (0, 0)),
            pl.BlockSpec((BLOCK_ROWS, HALF), lambda i: (i, 0)),
            pl.BlockSpec((BLOCK_ROWS, HALF), lambda i: (i, 0)),
            pl.BlockSpec((IN_F, OUT_F), lambda i: (0, 0)),
            pl.BlockSpec((1, OUT_F), lambda i: (0, 0)),
        ],
        out_specs=pl.BlockSpec((BLOCK_ROWS, OUT_F), lambda i: (i, 0)),
        out_shape=jax.ShapeDtypeStruct((N, OUT_F), jnp.float32),
        compiler_params=pltpu.CompilerParams(
            dimension_semantics=("parallel",),
        ),
    )(inp0, inp1, adj0, adj1, W, b2)


# speedup vs baseline: 2.2927x; 2.2927x over previous
import jax
import jax.numpy as jnp
from jax.experimental import pallas as pl
from jax.experimental.pallas import tpu as pltpu

N = 4096
IN_F = 64
OUT_F = 64
BLOCK_ROWS = 512


def _gcn_kernel(inp_ref, adj0_ref, adj1_ref, w_ref, b_ref, out_ref):
    t0 = jnp.dot(adj0_ref[...], inp_ref[...], preferred_element_type=jnp.float32)
    t1 = jnp.dot(adj1_ref[...], inp_ref[...], preferred_element_type=jnp.float32)
    out_ref[:BLOCK_ROWS, :] = (
        jnp.dot(t0, w_ref[...], preferred_element_type=jnp.float32) + b_ref[...]
    )
    out_ref[BLOCK_ROWS:, :] = (
        jnp.dot(t1, w_ref[...], preferred_element_type=jnp.float32) + b_ref[...]
    )


def kernel(input, adj, W, b):
    b2 = b.reshape(1, OUT_F)
    grid = (N // (2 * BLOCK_ROWS),)
    return pl.pallas_call(
        _gcn_kernel,
        grid=grid,
        in_specs=[
            pl.BlockSpec((N, IN_F), lambda i: (0, 0)),
            pl.BlockSpec((BLOCK_ROWS, N), lambda i: (2 * i, 0)),
            pl.BlockSpec((BLOCK_ROWS, N), lambda i: (2 * i + 1, 0)),
            pl.BlockSpec((IN_F, OUT_F), lambda i: (0, 0)),
            pl.BlockSpec((1, OUT_F), lambda i: (0, 0)),
        ],
        out_specs=pl.BlockSpec((2 * BLOCK_ROWS, OUT_F), lambda i: (i, 0)),
        out_shape=jax.ShapeDtypeStruct((N, OUT_F), jnp.float32),
        compiler_params=pltpu.CompilerParams(
            dimension_semantics=("parallel",),
        ),
    )(input, adj, adj, W, b2)


# speedup vs baseline: 2.4647x; 1.0750x over previous
import jax
import jax.numpy as jnp
from jax.experimental import pallas as pl
from jax.experimental.pallas import tpu as pltpu

N = 4096
IN_F = 64
OUT_F = 64
BLOCK_ROWS = 512


def _gcn_kernel(inp_ref, adj_ref, w_ref, b_ref, out_ref):
    t = jnp.dot(adj_ref[...], inp_ref[...], preferred_element_type=jnp.float32)
    out_ref[...] = (
        jnp.dot(t, w_ref[...], preferred_element_type=jnp.float32) + b_ref[...]
    )


def kernel(input, adj, W, b):
    b2 = b.reshape(1, OUT_F)
    grid = (N // BLOCK_ROWS,)
    return pl.pallas_call(
        _gcn_kernel,
        grid=grid,
        in_specs=[
            pl.BlockSpec((N, IN_F), lambda i: (0, 0)),
            pl.BlockSpec((BLOCK_ROWS, N), lambda i: (i, 0)),
            pl.BlockSpec((IN_F, OUT_F), lambda i: (0, 0)),
            pl.BlockSpec((1, OUT_F), lambda i: (0, 0)),
        ],
        out_specs=pl.BlockSpec((BLOCK_ROWS, OUT_F), lambda i: (i, 0)),
        out_shape=jax.ShapeDtypeStruct((N, OUT_F), jnp.float32),
        compiler_params=pltpu.CompilerParams(
            dimension_semantics=("parallel",),
        ),
    )(input, adj, W, b2)
